# Initial kernel scaffold; baseline (speedup 1.0000x reference)
import jax
import jax.numpy as jnp
from jax.experimental import pallas as pl


def kernel(x, poly_conn, poly_index, num_nodes, qkv_weight, qkv_bias, E_w, E_b, conn_lin_w, conn_lin_b, nodelin_w, nodelin_b, ln_h_w, ln_h_b, ln_e_w, ln_e_b):
    raise NotImplementedError("write your pallas kernel here")


# speedup vs baseline: 1.1004x; 1.1004x over previous
import functools

import jax
import jax.numpy as jnp
from jax import lax
from jax.experimental import pallas as pl
from jax.experimental.pallas import tpu as pltpu
from jax.experimental.pallas import tpu_sc as plsc

NC = 2
NS = 16
NW = NC * NS
L = 16


_GATHER_DNUMS = lax.GatherDimensionNumbers(
    offset_dims=(), collapsed_slice_dims=(0,), start_index_map=(0,))


def _lane_perm(x, idx):
    return lax.gather(x, idx[:, None], _GATHER_DNUMS, slice_sizes=(1,),
                      mode=lax.GatherScatterMode.PROMISE_IN_BOUNDS)


def _lane_sum(x):
    iota = lax.iota(jnp.int32, L)
    for sh in (8, 4, 2, 1):
        x = x + _lane_perm(x, iota ^ sh)
    return x


def _rsqrt_nr(v):
    i = lax.bitcast_convert_type(v, jnp.int32)
    i = jnp.int32(0x5F3759DF) - jnp.right_shift(i, jnp.int32(1))
    y = lax.bitcast_convert_type(i, jnp.float32)
    for _ in range(3):
        y = y * (1.5 - 0.5 * v * y * y)
    return y


def _proj_body(x_ref, wq, wk, wv, bq, bk, bv, q_ref, k_ref, v_ref):
    xb = x_ref[...]
    q_ref[...] = jnp.dot(xb, wq[...], preferred_element_type=jnp.float32) + bq[...]
    k_ref[...] = jnp.dot(xb, wk[...], preferred_element_type=jnp.float32) + bk[...]
    v_ref[...] = jnp.dot(xb, wv[...], preferred_element_type=jnp.float32) + bv[...]


def _eh_body(pc_ref, wT, b2, eh_ref):
    eh_ref[...] = jnp.dot(pc_ref[...], wT[...], preferred_element_type=jnp.float32) + b2[...]


def _final_body(aggp_ref, eaggp_ref, x_ref, clT, clb, nlT, nlb, lnw, lnb, h_ref):
    agg = aggp_ref[0] + aggp_ref[1]
    eagg = eaggp_ref[0] + eaggp_ref[1]
    el = jnp.dot(eagg, clT[...], preferred_element_type=jnp.float32) + clb[...]
    hsum = agg + el
    hm = jnp.dot(hsum, nlT[...], preferred_element_type=jnp.float32) + nlb[...]
    t = x_ref[...] + hm
    m = jnp.mean(t, axis=1, keepdims=True)
    var = jnp.mean((t - m) ** 2, axis=1, keepdims=True)
    h_ref[...] = (t - m) * lax.rsqrt(var + 1e-5) * lnw[...] + lnb[...]


def _sc_agg_body(src_hbm, dst_hbm, vh_hbm, zeros_hbm, outp_hbm,
                 idx_s, idx_d, rows_v, accum, sem, *, n_pad, per_w, c):
    cid = lax.axis_index("c")
    sid = lax.axis_index("s")
    wid = cid * NS + sid
    rows_pt = n_pad // NS
    pltpu.sync_copy(zeros_hbm.at[pl.ds(sid * rows_pt, rows_pt)],
                    accum.at[pl.ds(sid * rows_pt, rows_pt)])
    plsc.subcore_barrier()

    def chunk(j, carry):
        base = pl.multiple_of(wid * per_w + j * c, 8)
        pltpu.sync_copy(src_hbm.at[pl.ds(base, c)], idx_s)
        pltpu.sync_copy(dst_hbm.at[pl.ds(base, c)], idx_d)
        pltpu.async_copy(vh_hbm.at[idx_s], rows_v, sem).wait()
        pltpu.sync_copy(rows_v, accum.at[idx_d], add=True)
        return carry

    lax.fori_loop(0, per_w // c, chunk, 0)
    plsc.subcore_barrier()
    pltpu.sync_copy(accum.at[pl.ds(sid * rows_pt, rows_pt)],
                    outp_hbm.at[pl.ds(cid * n_pad + sid * rows_pt, rows_pt)])


def _sc_edge_body(dst_hbm, src_hbm, eh_hbm, pc_hbm, qh_hbm, kh_hbm, zeros_hbm,
                  lnw_hbm, lnb_hbm, e_hbm, outp_hbm,
                  idx_d, idx_s, eh_v, pc_v, q_v, k_v, act_v, e_v, lnw_v, lnb_v,
                  accum, sem, *, n_pad, d, per_w, c):
    cid = lax.axis_index("c")
    sid = lax.axis_index("s")
    wid = cid * NS + sid
    rows_pt = n_pad // NS
    ng = d // L
    pltpu.sync_copy(zeros_hbm.at[pl.ds(sid * rows_pt, rows_pt)],
                    accum.at[pl.ds(sid * rows_pt, rows_pt)])
    pltpu.sync_copy(lnw_hbm, lnw_v)
    pltpu.sync_copy(lnb_hbm, lnb_v)
    plsc.subcore_barrier()

    def chunk(j, carry):
        base = pl.multiple_of(wid * per_w + j * c, 8)
        pltpu.sync_copy(dst_hbm.at[pl.ds(base, c)], idx_d)
        pltpu.sync_copy(src_hbm.at[pl.ds(base, c)], idx_s)
        cp_q = pltpu.async_copy(qh_hbm.at[idx_d], q_v, sem)
        cp_k = pltpu.async_copy(kh_hbm.at[idx_s], k_v, sem)
        pltpu.sync_copy(eh_hbm.at[pl.ds(base, c)], eh_v)
        pltpu.sync_copy(pc_hbm.at[pl.ds(base, c)], pc_v)
        cp_q.wait()
        cp_k.wait()

        def row(r, carry2):
            svec = jnp.zeros((L,), jnp.float32)
            ssvec = jnp.zeros((L,), jnp.float32)
            ts = []
            for g in range(ng):
                sl = pl.ds(g * L, L)
                c1 = (q_v[r, sl] + k_v[r, sl]) * eh_v[r, sl]
                s2 = c1 * _rsqrt_nr(jnp.abs(c1))
                act = jnp.maximum(s2 + eh_v[r, pl.ds(d + g * L, L)], 0.0)
                act_v[r, sl] = act
                t = pc_v[r, sl] + act
                ts.append(t)
                svec = svec + t
                ssvec = ssvec + t * t
            mv = _lane_sum(svec) * (1.0 / d)
            var = jnp.maximum(_lane_sum(ssvec) * (1.0 / d) - mv * mv, 0.0)
            rstd = _rsqrt_nr(var + 1e-5)
            for g in range(ng):
                sl = pl.ds(g * L, L)
                e_v[r, sl] = (ts[g] - mv) * rstd * lnw_v[sl] + lnb_v[sl]
            return carry2

        lax.fori_loop(0, c, row, 0)
        pltpu.sync_copy(e_v, e_hbm.at[pl.ds(base, c)])
        pltpu.sync_copy(act_v, accum.at[idx_d], add=True)
        return carry

    lax.fori_loop(0, per_w // c, chunk, 0)
    plsc.subcore_barrier()
    pltpu.sync_copy(accum.at[pl.ds(sid * rows_pt, rows_pt)],
                    outp_hbm.at[pl.ds(cid * n_pad + sid * rows_pt, rows_pt)])


def kernel(x, poly_conn, poly_index, num_nodes, qkv_weight, qkv_bias, E_w, E_b,
           conn_lin_w, conn_lin_b, nodelin_w, nodelin_b, ln_h_w, ln_h_b,
           ln_e_w, ln_e_b):
    n, d = x.shape
    ne = poly_conn.shape[0]
    f32 = jnp.float32

    bn = 2000
    wqT = qkv_weight[:d].T
    wkT = qkv_weight[d:2 * d].T
    wvT = qkv_weight[2 * d:].T
    bq = qkv_bias[:d].reshape(1, d)
    bk = qkv_bias[d:2 * d].reshape(1, d)
    bv = qkv_bias[2 * d:].reshape(1, d)
    wspec = pl.BlockSpec((d, d), lambda i: (0, 0))
    bspec = pl.BlockSpec((1, d), lambda i: (0, 0))
    rowspec = pl.BlockSpec((bn, d), lambda i: (i, 0))
    qh, kh, vh = pl.pallas_call(
        _proj_body,
        grid=(n // bn,),
        in_specs=[rowspec, wspec, wspec, wspec, bspec, bspec, bspec],
        out_specs=[rowspec, rowspec, rowspec],
        out_shape=[jax.ShapeDtypeStruct((n, d), f32)] * 3,
    )(x, wqT, wkT, wvT, bq, bk, bv)

    be = 2000
    eh = pl.pallas_call(
        _eh_body,
        grid=(ne // be,),
        in_specs=[pl.BlockSpec((be, d), lambda i: (i, 0)),
                  pl.BlockSpec((d, 2 * d), lambda i: (0, 0)),
                  pl.BlockSpec((1, 2 * d), lambda i: (0, 0))],
        out_specs=pl.BlockSpec((be, 2 * d), lambda i: (i, 0)),
        out_shape=jax.ShapeDtypeStruct((ne, 2 * d), f32),
    )(poly_conn, E_w.T, E_b.reshape(1, 2 * d))

    dst = poly_index[0]
    src = poly_index[1]
    rows_pt = (-(-n // NS) + 7) // 8 * 8
    n_pad = NS * rows_pt
    zeros = jnp.zeros((n_pad, d), f32)
    per_w = ne // NW
    c = 40
    mesh = plsc.VectorSubcoreMesh(core_axis_name="c", subcore_axis_name="s")

    agg_fn = functools.partial(_sc_agg_body, n_pad=n_pad, per_w=per_w, c=c)
    aggp = pl.kernel(
        agg_fn, mesh=mesh,
        out_type=jax.ShapeDtypeStruct((NC * n_pad, d), f32),
        scratch_types=[
            pltpu.VMEM((c,), jnp.int32),
            pltpu.VMEM((c,), jnp.int32),
            pltpu.VMEM((c, d), f32),
            pltpu.VMEM_SHARED((n_pad, d), f32),
            pltpu.SemaphoreType.DMA,
        ],
    )(src, dst, vh, zeros)

    edge_fn = functools.partial(_sc_edge_body, n_pad=n_pad, d=d, per_w=per_w, c=c)
    e_out, eaggp = pl.kernel(
        edge_fn, mesh=mesh,
        out_type=[jax.ShapeDtypeStruct((ne, d), f32),
                  jax.ShapeDtypeStruct((NC * n_pad, d), f32)],
        scratch_types=[
            pltpu.VMEM((c,), jnp.int32),
            pltpu.VMEM((c,), jnp.int32),
            pltpu.VMEM((c, 2 * d), f32),
            pltpu.VMEM((c, d), f32),
            pltpu.VMEM((c, d), f32),
            pltpu.VMEM((c, d), f32),
            pltpu.VMEM((c, d), f32),
            pltpu.VMEM((c, d), f32),
            pltpu.VMEM((d,), f32),
            pltpu.VMEM((d,), f32),
            pltpu.VMEM_SHARED((n_pad, d), f32),
            pltpu.SemaphoreType.DMA,
        ],
    )(dst, src, eh, poly_conn, qh, kh, zeros, ln_e_w, ln_e_b)

    h = pl.pallas_call(
        _final_body,
        grid=(n // bn,),
        in_specs=[pl.BlockSpec((NC, bn, d), lambda i: (0, i, 0)),
                  pl.BlockSpec((NC, bn, d), lambda i: (0, i, 0)),
                  rowspec, wspec, bspec, wspec, bspec, bspec, bspec],
        out_specs=rowspec,
        out_shape=jax.ShapeDtypeStruct((n, d), f32),
    )(aggp.reshape(NC, n_pad, d)[:, :n], eaggp.reshape(NC, n_pad, d)[:, :n], x,
      conn_lin_w.T, conn_lin_b.reshape(1, d), nodelin_w.T,
      nodelin_b.reshape(1, d), ln_h_w.reshape(1, d), ln_h_b.reshape(1, d))

    return (h, e_out)


# speedup vs baseline: 2.0391x; 1.8530x over previous
import functools

import jax
import jax.numpy as jnp
from jax import lax
from jax.experimental import pallas as pl
from jax.experimental.pallas import tpu as pltpu
from jax.experimental.pallas import tpu_sc as plsc

NC = 2
NS = 16
NW = NC * NS
L = 16


_GATHER_DNUMS = lax.GatherDimensionNumbers(
    offset_dims=(), collapsed_slice_dims=(0,), start_index_map=(0,))


def _lane_perm(x, idx):
    return lax.gather(x, idx[:, None], _GATHER_DNUMS, slice_sizes=(1,),
                      mode=lax.GatherScatterMode.PROMISE_IN_BOUNDS)


def _lane_sum(x):
    iota = lax.iota(jnp.int32, L)
    for sh in (8, 4, 2, 1):
        x = x + _lane_perm(x, iota ^ sh)
    return x


def _rsqrt_nr(v):
    i = lax.bitcast_convert_type(v, jnp.int32)
    i = jnp.int32(0x5F3759DF) - jnp.right_shift(i, jnp.int32(1))
    y = lax.bitcast_convert_type(i, jnp.float32)
    for _ in range(2):
        y = y * (1.5 - 0.5 * v * y * y)
    return y


def _proj_body(x_ref, wq, wk, wv, bq, bk, bv, q_ref, k_ref, v_ref):
    xb = x_ref[...]
    q_ref[...] = jnp.dot(xb, wq[...], preferred_element_type=jnp.float32) + bq[...]
    k_ref[...] = jnp.dot(xb, wk[...], preferred_element_type=jnp.float32) + bk[...]
    v_ref[...] = jnp.dot(xb, wv[...], preferred_element_type=jnp.float32) + bv[...]


def _eh_body(pc_ref, wT, b2, eh_ref):
    eh_ref[...] = jnp.dot(pc_ref[...], wT[...], preferred_element_type=jnp.float32) + b2[...]


def _final_body(aggp_ref, eaggp_ref, x_ref, clT, clb, nlT, nlb, lnw, lnb, h_ref):
    agg = aggp_ref[0] + aggp_ref[1]
    eagg = eaggp_ref[0] + eaggp_ref[1]
    el = jnp.dot(eagg, clT[...], preferred_element_type=jnp.float32) + clb[...]
    hsum = agg + el
    hm = jnp.dot(hsum, nlT[...], preferred_element_type=jnp.float32) + nlb[...]
    t = x_ref[...] + hm
    m = jnp.mean(t, axis=1, keepdims=True)
    var = jnp.mean((t - m) ** 2, axis=1, keepdims=True)
    h_ref[...] = (t - m) * lax.rsqrt(var + 1e-5) * lnw[...] + lnb[...]


def _sc_agg_body(src_hbm, dst_hbm, vh_hbm, zeros_hbm, outp_hbm,
                 idx_s, idx_d, rows_v, accum, sem, *, n_pad, per_w, c):
    cid = lax.axis_index("c")
    sid = lax.axis_index("s")
    wid = cid * NS + sid
    rows_pt = n_pad // NS
    pltpu.sync_copy(zeros_hbm.at[pl.ds(sid * rows_pt, rows_pt)],
                    accum.at[pl.ds(sid * rows_pt, rows_pt)])
    plsc.subcore_barrier()

    def chunk(j, carry):
        base = pl.multiple_of(wid * per_w + j * c, 8)
        pltpu.sync_copy(src_hbm.at[pl.ds(base, c)], idx_s)
        pltpu.sync_copy(dst_hbm.at[pl.ds(base, c)], idx_d)
        pltpu.async_copy(vh_hbm.at[idx_s], rows_v, sem).wait()
        pltpu.sync_copy(rows_v, accum.at[idx_d], add=True)
        return carry

    lax.fori_loop(0, per_w // c, chunk, 0)
    plsc.subcore_barrier()
    pltpu.sync_copy(accum.at[pl.ds(sid * rows_pt, rows_pt)],
                    outp_hbm.at[pl.ds(cid * n_pad + sid * rows_pt, rows_pt)])


def _sc_edge_body(dst_hbm, src_hbm, eh_hbm, pc_hbm, qh_hbm, kh_hbm, zeros_hbm,
                  lnw_hbm, lnb_hbm, e_hbm, outp_hbm,
                  idx_d, idx_s, eh_v, pc_v, q_v, k_v, act_v, e_v, lnw_v, lnb_v,
                  accum, sem, *, n_pad, d, per_w, c):
    cid = lax.axis_index("c")
    sid = lax.axis_index("s")
    wid = cid * NS + sid
    rows_pt = n_pad // NS
    ng = d // L
    pltpu.sync_copy(zeros_hbm.at[pl.ds(sid * rows_pt, rows_pt)],
                    accum.at[pl.ds(sid * rows_pt, rows_pt)])
    pltpu.sync_copy(lnw_hbm, lnw_v)
    pltpu.sync_copy(lnb_hbm, lnb_v)
    plsc.subcore_barrier()

    def chunk(j, carry):
        base = pl.multiple_of(wid * per_w + j * c, 8)
        pltpu.sync_copy(dst_hbm.at[pl.ds(base, c)], idx_d)
        pltpu.sync_copy(src_hbm.at[pl.ds(base, c)], idx_s)
        cp_q = pltpu.async_copy(qh_hbm.at[idx_d], q_v, sem)
        cp_k = pltpu.async_copy(kh_hbm.at[idx_s], k_v, sem)
        pltpu.sync_copy(eh_hbm.at[pl.ds(base, c)], eh_v)
        pltpu.sync_copy(pc_hbm.at[pl.ds(base, c)], pc_v)
        cp_q.wait()
        cp_k.wait()

        @plsc.parallel_loop(0, c, 1, unroll=4)
        def row(r):
            svec = jnp.zeros((L,), jnp.float32)
            ssvec = jnp.zeros((L,), jnp.float32)
            ts = []
            for g in range(ng):
                sl = pl.ds(g * L, L)
                c1 = (q_v[r, sl] + k_v[r, sl]) * eh_v[r, sl]
                s2 = c1 * _rsqrt_nr(jnp.abs(c1))
                act = jnp.maximum(s2 + eh_v[r, pl.ds(d + g * L, L)], 0.0)
                act_v[r, sl] = act
                t = pc_v[r, sl] + act
                ts.append(t)
                svec = svec + t
                ssvec = ssvec + t * t
            mv = _lane_sum(svec) * (1.0 / d)
            var = jnp.maximum(_lane_sum(ssvec) * (1.0 / d) - mv * mv, 0.0)
            rstd = _rsqrt_nr(var + 1e-5)
            for g in range(ng):
                sl = pl.ds(g * L, L)
                e_v[r, sl] = (ts[g] - mv) * rstd * lnw_v[sl] + lnb_v[sl]

        pltpu.sync_copy(e_v, e_hbm.at[pl.ds(base, c)])
        pltpu.sync_copy(act_v, accum.at[idx_d], add=True)
        return carry

    lax.fori_loop(0, per_w // c, chunk, 0)
    plsc.subcore_barrier()
    pltpu.sync_copy(accum.at[pl.ds(sid * rows_pt, rows_pt)],
                    outp_hbm.at[pl.ds(cid * n_pad + sid * rows_pt, rows_pt)])


def kernel(x, poly_conn, poly_index, num_nodes, qkv_weight, qkv_bias, E_w, E_b,
           conn_lin_w, conn_lin_b, nodelin_w, nodelin_b, ln_h_w, ln_h_b,
           ln_e_w, ln_e_b):
    n, d = x.shape
    ne = poly_conn.shape[0]
    f32 = jnp.float32

    bn = 2000
    wqT = qkv_weight[:d].T
    wkT = qkv_weight[d:2 * d].T
    wvT = qkv_weight[2 * d:].T
    bq = qkv_bias[:d].reshape(1, d)
    bk = qkv_bias[d:2 * d].reshape(1, d)
    bv = qkv_bias[2 * d:].reshape(1, d)
    wspec = pl.BlockSpec((d, d), lambda i: (0, 0))
    bspec = pl.BlockSpec((1, d), lambda i: (0, 0))
    rowspec = pl.BlockSpec((bn, d), lambda i: (i, 0))
    qh, kh, vh = pl.pallas_call(
        _proj_body,
        grid=(n // bn,),
        in_specs=[rowspec, wspec, wspec, wspec, bspec, bspec, bspec],
        out_specs=[rowspec, rowspec, rowspec],
        out_shape=[jax.ShapeDtypeStruct((n, d), f32)] * 3,
    )(x, wqT, wkT, wvT, bq, bk, bv)

    be = 2000
    eh = pl.pallas_call(
        _eh_body,
        grid=(ne // be,),
        in_specs=[pl.BlockSpec((be, d), lambda i: (i, 0)),
                  pl.BlockSpec((d, 2 * d), lambda i: (0, 0)),
                  pl.BlockSpec((1, 2 * d), lambda i: (0, 0))],
        out_specs=pl.BlockSpec((be, 2 * d), lambda i: (i, 0)),
        out_shape=jax.ShapeDtypeStruct((ne, 2 * d), f32),
    )(poly_conn, E_w.T, E_b.reshape(1, 2 * d))

    dst = poly_index[0]
    src = poly_index[1]
    rows_pt = (-(-n // NS) + 7) // 8 * 8
    n_pad = NS * rows_pt
    zeros = jnp.zeros((n_pad, d), f32)
    per_w = ne // NW
    c = 40
    mesh = plsc.VectorSubcoreMesh(core_axis_name="c", subcore_axis_name="s")

    agg_fn = functools.partial(_sc_agg_body, n_pad=n_pad, per_w=per_w, c=c)
    aggp = pl.kernel(
        agg_fn, mesh=mesh,
        out_type=jax.ShapeDtypeStruct((NC * n_pad, d), f32),
        scratch_types=[
            pltpu.VMEM((c,), jnp.int32),
            pltpu.VMEM((c,), jnp.int32),
            pltpu.VMEM((c, d), f32),
            pltpu.VMEM_SHARED((n_pad, d), f32),
            pltpu.SemaphoreType.DMA,
        ],
    )(src, dst, vh, zeros)

    edge_fn = functools.partial(_sc_edge_body, n_pad=n_pad, d=d, per_w=per_w, c=c)
    e_out, eaggp = pl.kernel(
        edge_fn, mesh=mesh,
        out_type=[jax.ShapeDtypeStruct((ne, d), f32),
                  jax.ShapeDtypeStruct((NC * n_pad, d), f32)],
        scratch_types=[
            pltpu.VMEM((c,), jnp.int32),
            pltpu.VMEM((c,), jnp.int32),
            pltpu.VMEM((c, 2 * d), f32),
            pltpu.VMEM((c, d), f32),
            pltpu.VMEM((c, d), f32),
            pltpu.VMEM((c, d), f32),
            pltpu.VMEM((c, d), f32),
            pltpu.VMEM((c, d), f32),
            pltpu.VMEM((d,), f32),
            pltpu.VMEM((d,), f32),
            pltpu.VMEM_SHARED((n_pad, d), f32),
            pltpu.SemaphoreType.DMA,
        ],
    )(dst, src, eh, poly_conn, qh, kh, zeros, ln_e_w, ln_e_b)

    h = pl.pallas_call(
        _final_body,
        grid=(n // bn,),
        in_specs=[pl.BlockSpec((NC, bn, d), lambda i: (0, i, 0)),
                  pl.BlockSpec((NC, bn, d), lambda i: (0, i, 0)),
                  rowspec, wspec, bspec, wspec, bspec, bspec, bspec],
        out_specs=rowspec,
        out_shape=jax.ShapeDtypeStruct((n, d), f32),
    )(aggp.reshape(NC, n_pad, d)[:, :n], eaggp.reshape(NC, n_pad, d)[:, :n], x,
      conn_lin_w.T, conn_lin_b.reshape(1, d), nodelin_w.T,
      nodelin_b.reshape(1, d), ln_h_w.reshape(1, d), ln_h_b.reshape(1, d))

    return (h, e_out)
